# baseline (speedup 1.0000x reference)
import functools

import jax
import jax.numpy as jnp
from jax import lax
from jax.experimental import pallas as pl
from jax.experimental.pallas import tpu as pltpu
from jax.experimental.pallas import tpu_sc as plsc

N = 10000
E = 160000
D_IN = 256
H = 512
L = 5
G = 64
C = 2

NP = 10240
EP = 163840
BN = 512
NB = NP // BN
NTILES = 16
NW = 32
EPT = EP // NW
Q = NP // 4
CROWS = 162
CAP = CROWS * 32
BLKE = 32


@functools.lru_cache(maxsize=None)
def _make_partition():
    mesh = plsc.VectorSubcoreMesh(core_axis_name="c", subcore_axis_name="s")

    @functools.partial(
        pl.kernel,
        out_type=(
            jax.ShapeDtypeStruct((4 * NW, CAP), jnp.int32),
            jax.ShapeDtypeStruct((4 * NW, CAP), jnp.int32),
            jax.ShapeDtypeStruct((NW, 16), jnp.int32),
        ),
        mesh=mesh,
        scratch_types=[
            pltpu.VMEM((EPT,), jnp.int32),
            pltpu.VMEM((EPT,), jnp.int32),
            pltpu.VMEM((CAP,), jnp.int32),
            pltpu.VMEM((CAP,), jnp.int32),
            pltpu.VMEM((CAP,), jnp.int32),
            pltpu.VMEM((CAP,), jnp.int32),
            pltpu.VMEM((CAP,), jnp.int32),
            pltpu.VMEM((CAP,), jnp.int32),
            pltpu.VMEM((CAP,), jnp.int32),
            pltpu.VMEM((CAP,), jnp.int32),
            pltpu.VMEM((16,), jnp.int32),
        ],
        compiler_params=pltpu.CompilerParams(needs_layout_passes=False),
    )
    def part(src_hbm, dst_hbm, psrc_hbm, pdst_hbm, cnt_hbm,
             src_t, dst_t, ss0, ss1, ss2, ss3, sd0, sd1, sd2, sd3, cnt_v):
        ssrc = (ss0, ss1, ss2, ss3)
        sdst = (sd0, sd1, sd2, sd3)
        core = lax.axis_index("c")
        sub = lax.axis_index("s")
        w = sub * 2 + core
        pltpu.sync_copy(src_hbm.at[pl.ds(w * EPT, EPT)], src_t)
        pltpu.sync_copy(dst_hbm.at[pl.ds(w * EPT, EPT)], dst_t)

        one16 = jnp.ones((16,), jnp.int32)
        zero16 = jnp.zeros((16,), jnp.int32)

        def body(i, offs):
            vs = src_t[pl.ds(i * 16, 16)]
            vd = dst_t[pl.ds(i * 16, 16)]
            b = (jnp.where(vd >= Q, one16, zero16)
                 + jnp.where(vd >= 2 * Q, one16, zero16)
                 + jnp.where(vd >= 3 * Q, one16, zero16))
            new_offs = []
            for q in range(4):
                mask = b == q
                mi = jnp.where(mask, one16, zero16)
                pos = offs[q] + plsc.cumsum(mi) - 1
                plsc.store_scatter(ssrc[q], [pos], vs, mask=mask)
                plsc.store_scatter(sdst[q], [pos], vd - q * Q, mask=mask)
                new_offs.append(offs[q] + jnp.sum(mi))
            return tuple(new_offs)

        offs = lax.fori_loop(0, EPT // 16, body,
                             (jnp.int32(0), jnp.int32(0),
                              jnp.int32(0), jnp.int32(0)))

        lane = lax.broadcasted_iota(jnp.int32, (16,), 0)
        zeros16 = jnp.zeros((16,), jnp.int32)
        dummy16 = jnp.full((16,), Q, jnp.int32)
        cvec = jnp.zeros((16,), jnp.int32)
        for q in range(4):
            for extra in (0, 16):
                pos = offs[q] + extra + lane
            cvec = jnp.where(lane == q,
                             jnp.broadcast_to(offs[q], (16,)), cvec)
        cnt_v[...] = cvec
        pltpu.sync_copy(cnt_v, cnt_hbm.at[w])
        for q in range(4):
            pltpu.sync_copy(ssrc[q], psrc_hbm.at[q * NW + w])
            pltpu.sync_copy(sdst[q], pdst_hbm.at[q * NW + w])

    return part


@functools.lru_cache(maxsize=None)
def _make_sc_segsum(D):
    mesh = plsc.VectorSubcoreMesh(core_axis_name="c", subcore_axis_name="s")
    ZROWS = 32
    RPT = Q // NTILES
    R = D // 128

    @functools.partial(
        pl.kernel,
        out_type=jax.ShapeDtypeStruct((NP, R, 128), jnp.float32),
        mesh=mesh,
        scratch_types=[
            pltpu.VMEM_SHARED((Q + 8, R, 128), jnp.float32),
            pltpu.VMEM((CAP,), jnp.int32),
            pltpu.VMEM((CAP,), jnp.int32),
            pltpu.VMEM((2, 16), jnp.int32),
            pltpu.VMEM((2, BLKE, R, 128), jnp.float32),
            pltpu.VMEM((2, BLKE), jnp.int32),
            pltpu.VMEM((2, BLKE), jnp.int32),
            pltpu.SemaphoreType.DMA,
            pltpu.SemaphoreType.DMA,
            pltpu.SemaphoreType.DMA,
        ],
    )
    def segsum(table_hbm, psrc_hbm, pdst_hbm, cnt_hbm, zeros_hbm, out_hbm,
               acc_sp, src_t, dst_t, cnt_v, rows_v, gidx_v, didx_v, gsem0, gsem1, ssem):
        core = lax.axis_index("c")
        sub = lax.axis_index("s")
        row0 = sub * RPT
        gsems = (gsem0, gsem1)

        def gather(p):
            return pltpu.make_async_copy(
                table_hbm.at[gidx_v.at[p]], rows_v.at[p], gsems[p])

        def stage(buf, p, flat, t):
            for i in range(BLKE // 16):
                buf[p, pl.ds(i * 16, 16)] = flat[pl.ds(t * BLKE + i * 16, 16)]

        pltpu.sync_copy(cnt_hbm.at[pl.ds(sub * 2, 2)], cnt_v)
        for qi in range(2):
            q = 2 * core + qi
            for z in range(RPT // ZROWS):
                pltpu.sync_copy(
                    zeros_hbm, acc_sp.at[pl.ds(row0 + z * ZROWS, ZROWS)])
            plsc.subcore_barrier()
            for ri in range(2):
                reg = q * NW + sub * 2 + ri
                pltpu.sync_copy(psrc_hbm.at[reg], src_t)
                pltpu.sync_copy(pdst_hbm.at[reg], dst_t)
                cv = cnt_v[ri, pl.ds(0, 16)]
                cnt = cv[qi] * (1 - core) + cv[2 + qi] * core
                nt = (cnt + BLKE - 1) // BLKE

                @pl.when(nt > 0)
                def _():
                    stage(gidx_v, 0, src_t, 0)
                    gather(0).start()

                    def body(t, carry):
                        even = lax.rem(t, 2) == 0

                        @pl.when(even)
                        def _():
                            gather(0).wait()

                            @pl.when(t + 1 < nt)
                            def _():
                                stage(gidx_v, 1, src_t, t + 1)
                                gather(1).start()
                            stage(didx_v, 0, dst_t, t)
                            d = pltpu.make_async_copy(
                                rows_v.at[0], acc_sp.at[didx_v.at[0]], ssem)
                            d.start(add=True)
                            d.wait()

                        @pl.when(lax.rem(t, 2) == 1)
                        def _():
                            gather(1).wait()

                            @pl.when(t + 1 < nt)
                            def _():
                                stage(gidx_v, 0, src_t, t + 1)
                                gather(0).start()
                            stage(didx_v, 1, dst_t, t)
                            d = pltpu.make_async_copy(
                                rows_v.at[1], acc_sp.at[didx_v.at[1]], ssem)
                            d.start(add=True)
                            d.wait()
                        return carry

                    lax.fori_loop(0, nt, body, 0)
            plsc.subcore_barrier()
            pltpu.sync_copy(
                acc_sp.at[pl.ds(row0, RPT)],
                out_hbm.at[pl.ds(q * Q + row0, RPT)])

    return segsum


def _mlp_body(h_ref, m_ref, w1_ref, b1_ref, w2_ref, b2_ref, o_ref):
    a = h_ref[...] + m_ref[...]
    t = jnp.maximum(
        jnp.dot(a, w1_ref[...], preferred_element_type=jnp.float32)
        + b1_ref[...], 0.0)
    o_ref[...] = jnp.maximum(
        jnp.dot(t, w2_ref[...], preferred_element_type=jnp.float32)
        + b2_ref[...], 0.0)


def _mlp_layer(h, msg, W1, b1, W2, b2):
    din = h.shape[1]
    return pl.pallas_call(
        _mlp_body,
        grid=(NB,),
        in_specs=[
            pl.BlockSpec((BN, din), lambda i: (i, 0)),
            pl.BlockSpec((BN, din), lambda i: (i, 0)),
            pl.BlockSpec((din, H), lambda i: (0, 0)),
            pl.BlockSpec((1, H), lambda i: (0, 0)),
            pl.BlockSpec((H, H), lambda i: (0, 0)),
            pl.BlockSpec((1, H), lambda i: (0, 0)),
        ],
        out_specs=pl.BlockSpec((BN, H), lambda i: (i, 0)),
        out_shape=jax.ShapeDtypeStruct((NP, H), jnp.float32),
    )(h, msg, W1, b1.reshape(1, H), W2, b2.reshape(1, H))


def _final_body(h_ref, m_ref, w1_ref, b1_ref, w2_ref, b2_ref, wc_ref, bc_ref,
                batch_ref, o_ref, pooled_acc, cnt_acc):
    i = pl.program_id(0)

    @pl.when(i == 0)
    def _():
        pooled_acc[...] = jnp.zeros_like(pooled_acc)
        cnt_acc[...] = jnp.zeros_like(cnt_acc)

    a = h_ref[...] + m_ref[...]
    t = jnp.maximum(
        jnp.dot(a, w1_ref[...], preferred_element_type=jnp.float32)
        + b1_ref[...], 0.0)
    t2 = jnp.maximum(
        jnp.dot(t, w2_ref[...], preferred_element_type=jnp.float32)
        + b2_ref[...], 0.0)

    b = batch_ref[0]
    gid = lax.broadcasted_iota(jnp.int32, (G, BN), 0)
    onehot = jnp.where(gid == jnp.broadcast_to(b, (G, BN)), 1.0, 0.0)
    pooled_acc[...] += jnp.dot(onehot, t2, preferred_element_type=jnp.float32)
    cnt_acc[...] += jnp.dot(onehot, jnp.ones((BN, 128), jnp.float32),
                            preferred_element_type=jnp.float32)

    @pl.when(i == NB - 1)
    def _():
        inv = 1.0 / jnp.maximum(cnt_acc[...], 1.0)
        scale = jnp.concatenate([inv] * 4, axis=1)
        pooled = pooled_acc[...] * scale
        o_ref[...] = (
            jnp.dot(pooled, wc_ref[...], preferred_element_type=jnp.float32)
            + bc_ref[...])


def _final_layer(h, msg, W1, b1, W2, b2, Wc_pad, bc_pad, batch3):
    return pl.pallas_call(
        _final_body,
        grid=(NB,),
        in_specs=[
            pl.BlockSpec((BN, H), lambda i: (i, 0)),
            pl.BlockSpec((BN, H), lambda i: (i, 0)),
            pl.BlockSpec((H, H), lambda i: (0, 0)),
            pl.BlockSpec((1, H), lambda i: (0, 0)),
            pl.BlockSpec((H, H), lambda i: (0, 0)),
            pl.BlockSpec((1, H), lambda i: (0, 0)),
            pl.BlockSpec((H, 128), lambda i: (0, 0)),
            pl.BlockSpec((1, 128), lambda i: (0, 0)),
            pl.BlockSpec((1, 1, BN), lambda i: (i, 0, 0)),
        ],
        out_specs=pl.BlockSpec((G, 128), lambda i: (0, 0)),
        out_shape=jax.ShapeDtypeStruct((G, 128), jnp.float32),
        scratch_shapes=[
            pltpu.VMEM((G, H), jnp.float32),
            pltpu.VMEM((G, 128), jnp.float32),
        ],
    )(h, msg, W1, b1.reshape(1, H), W2, b2.reshape(1, H),
      Wc_pad, bc_pad, batch3)


def kernel(x, edge_index, batch,
           W1_0, b1_0, W2_0, b2_0,
           W1_1, b1_1, W2_1, b2_1,
           W1_2, b1_2, W2_2, b2_2,
           W1_3, b1_3, W2_3, b2_3,
           W1_4, b1_4, W2_4, b2_4,
           Wc, bc):
    layers = [(W1_0, b1_0, W2_0, b2_0), (W1_1, b1_1, W2_1, b2_1),
              (W1_2, b1_2, W2_2, b2_2), (W1_3, b1_3, W2_3, b2_3),
              (W1_4, b1_4, W2_4, b2_4)]

    h = jnp.pad(x, ((0, NP - N), (0, 0)))
    src = jnp.concatenate([edge_index[0], jnp.zeros((EP - E,), jnp.int32)])
    dst = jnp.concatenate([edge_index[1], jnp.full((EP - E,), NP - 1,
                                                   jnp.int32)])
    zeros_hbm = jnp.zeros((32, H), jnp.float32)
    batch3 = jnp.concatenate(
        [batch, jnp.full((NP - N,), G, jnp.int32)]).reshape(NB, 1, BN)
    Wc_pad = jnp.pad(Wc, ((0, 0), (0, 128 - C)))
    bc_pad = jnp.pad(bc, ((0, 128 - C),)).reshape(1, 128)

    def _stub_partition(src, dst):
        srcw = src.reshape(NW, EPT)
        dstw = dst.reshape(NW, EPT)
        bw = dstw // Q
        order = jnp.argsort(bw, axis=1, stable=True)
        sb = jnp.take_along_axis(bw, order, axis=1)
        ss = jnp.take_along_axis(srcw, order, axis=1)
        sd = jnp.take_along_axis(dstw, order, axis=1) - sb * Q
        cnts_wq = jnp.sum(bw[:, :, None] == jnp.arange(4)[None, None, :], axis=1)
        starts = jnp.concatenate(
            [jnp.zeros((NW, 1), jnp.int32), jnp.cumsum(cnts_wq, axis=1)[:, :3].astype(jnp.int32)],
            axis=1)
        ar = jnp.arange(EPT)[None, :]
        local_pos = ar - jnp.take_along_axis(starts, sb, axis=1)
        flat_idx = sb * CAP + local_pos
        psrc = jnp.full((NW, 4 * CAP), 0, jnp.int32)
        pdst = jnp.full((NW, 4 * CAP), Q, jnp.int32)
        psrc = jax.vmap(lambda a, i, v: a.at[i].set(v))(psrc, flat_idx, ss)
        pdst = jax.vmap(lambda a, i, v: a.at[i].set(v))(pdst, flat_idx, sd)
        psrc = psrc.reshape(NW, 4, CAP).transpose(1, 0, 2).reshape(4 * NW, CAP)
        pdst = pdst.reshape(NW, 4, CAP).transpose(1, 0, 2).reshape(4 * NW, CAP)
        cnts = jnp.pad(cnts_wq.astype(jnp.int32), ((0, 0), (0, 12)))
        return psrc, pdst, cnts
    psrc, pdst, cnts = _stub_partition(src, dst)

    for l in range(L):
        W1, b1, W2, b2 = layers[l]
        D = h.shape[1]
        msg = _make_sc_segsum(D)(h.reshape(NP, D // 128, 128), psrc, pdst,
                                 cnts, zeros_hbm[:, :D].reshape(32, D // 128, 128)
                                 ).reshape(NP, D)
        if l < L - 1:
            h = _mlp_layer(h, msg, W1, b1, W2, b2)
        else:
            logits = _final_layer(h, msg, W1, b1, W2, b2,
                                  Wc_pad, bc_pad, batch3)
    return logits[:, :C]


# speedup vs baseline: 2.2634x; 2.2634x over previous
import functools

import jax
import jax.numpy as jnp
from jax import lax
from jax.experimental import pallas as pl
from jax.experimental.pallas import tpu as pltpu
from jax.experimental.pallas import tpu_sc as plsc

N = 10000
E = 160000
D_IN = 256
H = 512
L = 5
G = 64
C = 2

NP = 10240
EP = 163840
BN = 512
NB = NP // BN
NTILES = 16
ROWS_PER_TILE = NP // NTILES
ZR = 160
EROWS_PER_TILE = EP // NTILES // 128


BLK = 128
NSLOT = 2
BROWS_PER_TILE = EP // NTILES // BLK
NG = BROWS_PER_TILE // NSLOT


@functools.lru_cache(maxsize=None)
def _make_sc_segsum(nchunks):
    cpc = nchunks // 2
    mesh = plsc.VectorSubcoreMesh(core_axis_name="c", subcore_axis_name="s")

    @functools.partial(
        pl.kernel,
        out_type=jax.ShapeDtypeStruct((nchunks * NP, 128), jnp.float32),
        mesh=mesh,
        scratch_types=[
            pltpu.VMEM_SHARED((NP, 128), jnp.float32),
            pltpu.VMEM((2, NSLOT, BLK), jnp.int32),
            pltpu.VMEM((2, NSLOT, BLK), jnp.int32),
            pltpu.VMEM((NSLOT, BLK), jnp.int32),
            pltpu.VMEM((NSLOT, BLK, 128), jnp.float32),
        ] + [pltpu.SemaphoreType.DMA] * (2 * NSLOT + 4),
    )
    def segsum(table_hbm, src_hbm, dst_hbm, zeros_hbm, out_hbm,
               msg_sp, src_b, dst_b, gidx_v, rows_v, *sems):
        gsem = sems[:NSLOT]
        ssem = sems[NSLOT:2 * NSLOT]
        isem_s = sems[2 * NSLOT:2 * NSLOT + 2]
        isem_d = sems[2 * NSLOT + 2:]
        core = lax.axis_index("c")
        sub = lax.axis_index("s")
        row0 = sub * ROWS_PER_TILE
        brow0 = sub * BROWS_PER_TILE

        def idx_load(p, g):
            return (pltpu.make_async_copy(
                        src_hbm.at[pl.ds(brow0 + g * NSLOT, NSLOT)],
                        src_b.at[p], isem_s[p]),
                    pltpu.make_async_copy(
                        dst_hbm.at[pl.ds(brow0 + g * NSLOT, NSLOT)],
                        dst_b.at[p], isem_d[p]))

        def compute_gidx(p, j, off):
            for i in range(BLK // 16):
                gidx_v[j, pl.ds(i * 16, 16)] = (
                    src_b[p, j, pl.ds(i * 16, 16)] + off)

        def gather(j):
            return pltpu.make_async_copy(
                table_hbm.at[gidx_v.at[j]], rows_v.at[j], gsem[j])

        def scatter(p, j):
            return pltpu.make_async_copy(
                rows_v.at[j], msg_sp.at[dst_b.at[p, j]], ssem[j])

        for k in range(cpc):
            chunk = core * cpc + k
            off = chunk * NP
            for d in idx_load(0, 0):
                d.start()
            for d in idx_load(0, 0):
                d.wait()
            for j in range(NSLOT):
                compute_gidx(0, j, off)
                gather(j).start()
            for d in idx_load(1, 1):
                d.start()
            for j in range(ROWS_PER_TILE // ZR):
                pltpu.sync_copy(zeros_hbm, msg_sp.at[pl.ds(row0 + j * ZR, ZR)])
            plsc.subcore_barrier()

            def body(u, carry):
                for p in range(2):
                    g = 2 * u + p
                    for j in range(NSLOT):
                        gather(j).wait()
                        scatter(p, j).start(add=True)

                    @pl.when(g < NG - 1)
                    def _(p=p, g=g):
                        q = 1 - p
                        for d in idx_load(q, g + 1):
                            d.wait()
                        for j in range(NSLOT):
                            compute_gidx(q, j, off)
                            scatter(p, j).wait()
                            gather(j).start()

                        @pl.when(g < NG - 2)
                        def _():
                            for d in idx_load(p, g + 2):
                                d.start()
                return carry

            lax.fori_loop(0, NG // 2, body, 0)
            for j in range(NSLOT):
                scatter(1, j).wait()
            plsc.subcore_barrier()
            pltpu.sync_copy(
                msg_sp.at[pl.ds(row0, ROWS_PER_TILE)],
                out_hbm.at[pl.ds(off + row0, ROWS_PER_TILE)])

    return segsum


def _mlp_body(nc_in, h_ref, m_ref, w1_ref, b1_ref, w2_ref, b2_ref, o_ref):
    h = jnp.concatenate([h_ref[c] for c in range(nc_in)], axis=1)
    m = jnp.concatenate([m_ref[c] for c in range(nc_in)], axis=1)
    a = h + m
    t = jnp.maximum(
        jnp.dot(a, w1_ref[...], preferred_element_type=jnp.float32)
        + b1_ref[...], 0.0)
    t2 = jnp.maximum(
        jnp.dot(t, w2_ref[...], preferred_element_type=jnp.float32)
        + b2_ref[...], 0.0)
    for c in range(4):
        o_ref[c] = t2[:, c * 128:(c + 1) * 128]


def _mlp_layer(h3, msg3, W1, b1, W2, b2):
    nc_in = h3.shape[0]
    din = nc_in * 128
    return pl.pallas_call(
        functools.partial(_mlp_body, nc_in),
        grid=(NB,),
        in_specs=[
            pl.BlockSpec((nc_in, BN, 128), lambda i: (0, i, 0)),
            pl.BlockSpec((nc_in, BN, 128), lambda i: (0, i, 0)),
            pl.BlockSpec((din, H), lambda i: (0, 0)),
            pl.BlockSpec((1, H), lambda i: (0, 0)),
            pl.BlockSpec((H, H), lambda i: (0, 0)),
            pl.BlockSpec((1, H), lambda i: (0, 0)),
        ],
        out_specs=pl.BlockSpec((4, BN, 128), lambda i: (0, i, 0)),
        out_shape=jax.ShapeDtypeStruct((4, NP, 128), jnp.float32),
    )(h3, msg3, W1, b1.reshape(1, H), W2, b2.reshape(1, H))


def _final_body(h_ref, m_ref, w1_ref, b1_ref, w2_ref, b2_ref, wc_ref, bc_ref,
                batch_ref, o_ref, pooled_acc, cnt_acc):
    i = pl.program_id(0)

    @pl.when(i == 0)
    def _():
        pooled_acc[...] = jnp.zeros_like(pooled_acc)
        cnt_acc[...] = jnp.zeros_like(cnt_acc)

    h = jnp.concatenate([h_ref[c] for c in range(4)], axis=1)
    m = jnp.concatenate([m_ref[c] for c in range(4)], axis=1)
    a = h + m
    t = jnp.maximum(
        jnp.dot(a, w1_ref[...], preferred_element_type=jnp.float32)
        + b1_ref[...], 0.0)
    t2 = jnp.maximum(
        jnp.dot(t, w2_ref[...], preferred_element_type=jnp.float32)
        + b2_ref[...], 0.0)

    b = batch_ref[0]
    gid = lax.broadcasted_iota(jnp.int32, (G, BN), 0)
    onehot = jnp.where(gid == jnp.broadcast_to(b, (G, BN)), 1.0, 0.0)
    pooled_acc[...] += jnp.dot(onehot, t2, preferred_element_type=jnp.float32)
    cnt_acc[...] += jnp.dot(onehot, jnp.ones((BN, 128), jnp.float32),
                            preferred_element_type=jnp.float32)

    @pl.when(i == NB - 1)
    def _():
        inv = 1.0 / jnp.maximum(cnt_acc[...], 1.0)
        scale = jnp.concatenate([inv] * 4, axis=1)
        pooled = pooled_acc[...] * scale
        o_ref[...] = (
            jnp.dot(pooled, wc_ref[...], preferred_element_type=jnp.float32)
            + bc_ref[...])


def _final_layer(h3, msg3, W1, b1, W2, b2, Wc_pad, bc_pad, batch3):
    return pl.pallas_call(
        _final_body,
        grid=(NB,),
        in_specs=[
            pl.BlockSpec((4, BN, 128), lambda i: (0, i, 0)),
            pl.BlockSpec((4, BN, 128), lambda i: (0, i, 0)),
            pl.BlockSpec((H, H), lambda i: (0, 0)),
            pl.BlockSpec((1, H), lambda i: (0, 0)),
            pl.BlockSpec((H, H), lambda i: (0, 0)),
            pl.BlockSpec((1, H), lambda i: (0, 0)),
            pl.BlockSpec((H, 128), lambda i: (0, 0)),
            pl.BlockSpec((1, 128), lambda i: (0, 0)),
            pl.BlockSpec((1, 1, BN), lambda i: (i, 0, 0)),
        ],
        out_specs=pl.BlockSpec((G, 128), lambda i: (0, 0)),
        out_shape=jax.ShapeDtypeStruct((G, 128), jnp.float32),
        scratch_shapes=[
            pltpu.VMEM((G, H), jnp.float32),
            pltpu.VMEM((G, 128), jnp.float32),
        ],
    )(h3, msg3, W1, b1.reshape(1, H), W2, b2.reshape(1, H),
      Wc_pad, bc_pad, batch3)


def kernel(x, edge_index, batch,
           W1_0, b1_0, W2_0, b2_0,
           W1_1, b1_1, W2_1, b2_1,
           W1_2, b1_2, W2_2, b2_2,
           W1_3, b1_3, W2_3, b2_3,
           W1_4, b1_4, W2_4, b2_4,
           Wc, bc):
    layers = [(W1_0, b1_0, W2_0, b2_0), (W1_1, b1_1, W2_1, b2_1),
              (W1_2, b1_2, W2_2, b2_2), (W1_3, b1_3, W2_3, b2_3),
              (W1_4, b1_4, W2_4, b2_4)]

    x_pad = jnp.pad(x, ((0, NP - N), (0, 0)))
    h3 = jnp.transpose(x_pad.reshape(NP, 2, 128), (1, 0, 2))
    src = jnp.concatenate(
        [edge_index[0], jnp.zeros((EP - E,), jnp.int32)]).reshape(EP // BLK, BLK)
    dst = jnp.concatenate(
        [edge_index[1], jnp.full((EP - E,), NP - 1, jnp.int32)]).reshape(EP // BLK, BLK)
    zeros_hbm = jnp.zeros((ZR, 128), jnp.float32)
    batch3 = jnp.concatenate(
        [batch, jnp.full((NP - N,), G, jnp.int32)]).reshape(NB, 1, BN)
    Wc_pad = jnp.pad(Wc, ((0, 0), (0, 128 - C)))
    bc_pad = jnp.pad(bc, ((0, 128 - C),)).reshape(1, 128)

    for l in range(L):
        W1, b1, W2, b2 = layers[l]
        nc = h3.shape[0]
        msg = _make_sc_segsum(nc)(h3.reshape(nc * NP, 128), src, dst, zeros_hbm)
        msg3 = msg.reshape(nc, NP, 128)
        if l < L - 1:
            h3 = _mlp_layer(h3, msg3, W1, b1, W2, b2)
        else:
            logits = _final_layer(h3, msg3, W1, b1, W2, b2,
                                  Wc_pad, bc_pad, batch3)
    return logits[:, :C]


# speedup vs baseline: 2.4131x; 1.0661x over previous
import functools

import jax
import jax.numpy as jnp
from jax import lax
from jax.experimental import pallas as pl
from jax.experimental.pallas import tpu as pltpu
from jax.experimental.pallas import tpu_sc as plsc

N = 10000
E = 160000
D_IN = 256
H = 512
L = 5
G = 64
C = 2

NP = 10240
EP = 163840
BN = 512
NB = NP // BN
NTILES = 16
ROWS_PER_TILE = NP // NTILES
ZR = 160
EROWS_PER_TILE = EP // NTILES // 128


BLK = 64
NSLOT = 4
BROWS_PER_TILE = EP // NTILES // BLK
NG = BROWS_PER_TILE // NSLOT


@functools.lru_cache(maxsize=None)
def _make_sc_segsum(nchunks):
    cpc = nchunks // 2
    mesh = plsc.VectorSubcoreMesh(core_axis_name="c", subcore_axis_name="s")

    @functools.partial(
        pl.kernel,
        out_type=jax.ShapeDtypeStruct((nchunks * NP, 128), jnp.float32),
        mesh=mesh,
        scratch_types=[
            pltpu.VMEM_SHARED((NP, 128), jnp.float32),
            pltpu.VMEM((2, NSLOT, BLK), jnp.int32),
            pltpu.VMEM((2, NSLOT, BLK), jnp.int32),
            pltpu.VMEM((NSLOT, BLK), jnp.int32),
            pltpu.VMEM((NSLOT, BLK, 128), jnp.float32),
        ] + [pltpu.SemaphoreType.DMA] * (2 * NSLOT + 4),
    )
    def segsum(table_hbm, src_hbm, dst_hbm, zeros_hbm, out_hbm,
               msg_sp, src_b, dst_b, gidx_v, rows_v, *sems):
        gsem = sems[:NSLOT]
        ssem = sems[NSLOT:2 * NSLOT]
        isem_s = sems[2 * NSLOT:2 * NSLOT + 2]
        isem_d = sems[2 * NSLOT + 2:]
        core = lax.axis_index("c")
        sub = lax.axis_index("s")
        row0 = sub * ROWS_PER_TILE
        brow0 = sub * BROWS_PER_TILE

        def idx_load(p, g):
            return (pltpu.make_async_copy(
                        src_hbm.at[pl.ds(brow0 + g * NSLOT, NSLOT)],
                        src_b.at[p], isem_s[p]),
                    pltpu.make_async_copy(
                        dst_hbm.at[pl.ds(brow0 + g * NSLOT, NSLOT)],
                        dst_b.at[p], isem_d[p]))

        def compute_gidx(p, j, off):
            for i in range(BLK // 16):
                gidx_v[j, pl.ds(i * 16, 16)] = (
                    src_b[p, j, pl.ds(i * 16, 16)] + off)

        def gather(j):
            return pltpu.make_async_copy(
                table_hbm.at[gidx_v.at[j]], rows_v.at[j], gsem[j])

        def scatter(p, j):
            return pltpu.make_async_copy(
                rows_v.at[j], msg_sp.at[dst_b.at[p, j]], ssem[j])

        for k in range(cpc):
            chunk = core * cpc + k
            off = chunk * NP
            for d in idx_load(0, 0):
                d.start()
            for d in idx_load(0, 0):
                d.wait()
            for j in range(NSLOT):
                compute_gidx(0, j, off)
                gather(j).start()
            for d in idx_load(1, 1):
                d.start()
            for j in range(ROWS_PER_TILE // ZR):
                pltpu.sync_copy(zeros_hbm, msg_sp.at[pl.ds(row0 + j * ZR, ZR)])
            plsc.subcore_barrier()

            def body(u, carry):
                for p in range(2):
                    g = 2 * u + p
                    for j in range(NSLOT):
                        gather(j).wait()
                        scatter(p, j).start(add=True)

                    @pl.when(g < NG - 1)
                    def _(p=p, g=g):
                        q = 1 - p
                        for d in idx_load(q, g + 1):
                            d.wait()
                        for j in range(NSLOT):
                            compute_gidx(q, j, off)
                            scatter(p, j).wait()
                            gather(j).start()

                        @pl.when(g < NG - 2)
                        def _():
                            for d in idx_load(p, g + 2):
                                d.start()
                return carry

            lax.fori_loop(0, NG // 2, body, 0)
            for j in range(NSLOT):
                scatter(1, j).wait()
            plsc.subcore_barrier()
            pltpu.sync_copy(
                msg_sp.at[pl.ds(row0, ROWS_PER_TILE)],
                out_hbm.at[pl.ds(off + row0, ROWS_PER_TILE)])

    return segsum


def _mlp_body(nc_in, h_ref, m_ref, w1_ref, b1_ref, w2_ref, b2_ref, o_ref):
    h = jnp.concatenate([h_ref[c] for c in range(nc_in)], axis=1)
    m = jnp.concatenate([m_ref[c] for c in range(nc_in)], axis=1)
    a = h + m
    t = jnp.maximum(
        jnp.dot(a, w1_ref[...], preferred_element_type=jnp.float32)
        + b1_ref[...], 0.0)
    t2 = jnp.maximum(
        jnp.dot(t, w2_ref[...], preferred_element_type=jnp.float32)
        + b2_ref[...], 0.0)
    for c in range(4):
        o_ref[c] = t2[:, c * 128:(c + 1) * 128]


def _mlp_layer(h3, msg3, W1, b1, W2, b2):
    nc_in = h3.shape[0]
    din = nc_in * 128
    return pl.pallas_call(
        functools.partial(_mlp_body, nc_in),
        grid=(NB,),
        in_specs=[
            pl.BlockSpec((nc_in, BN, 128), lambda i: (0, i, 0)),
            pl.BlockSpec((nc_in, BN, 128), lambda i: (0, i, 0)),
            pl.BlockSpec((din, H), lambda i: (0, 0)),
            pl.BlockSpec((1, H), lambda i: (0, 0)),
            pl.BlockSpec((H, H), lambda i: (0, 0)),
            pl.BlockSpec((1, H), lambda i: (0, 0)),
        ],
        out_specs=pl.BlockSpec((4, BN, 128), lambda i: (0, i, 0)),
        out_shape=jax.ShapeDtypeStruct((4, NP, 128), jnp.float32),
    )(h3, msg3, W1, b1.reshape(1, H), W2, b2.reshape(1, H))


def _final_body(h_ref, m_ref, w1_ref, b1_ref, w2_ref, b2_ref, wc_ref, bc_ref,
                batch_ref, o_ref, pooled_acc, cnt_acc):
    i = pl.program_id(0)

    @pl.when(i == 0)
    def _():
        pooled_acc[...] = jnp.zeros_like(pooled_acc)
        cnt_acc[...] = jnp.zeros_like(cnt_acc)

    h = jnp.concatenate([h_ref[c] for c in range(4)], axis=1)
    m = jnp.concatenate([m_ref[c] for c in range(4)], axis=1)
    a = h + m
    t = jnp.maximum(
        jnp.dot(a, w1_ref[...], preferred_element_type=jnp.float32)
        + b1_ref[...], 0.0)
    t2 = jnp.maximum(
        jnp.dot(t, w2_ref[...], preferred_element_type=jnp.float32)
        + b2_ref[...], 0.0)

    b = batch_ref[0]
    gid = lax.broadcasted_iota(jnp.int32, (G, BN), 0)
    onehot = jnp.where(gid == jnp.broadcast_to(b, (G, BN)), 1.0, 0.0)
    pooled_acc[...] += jnp.dot(onehot, t2, preferred_element_type=jnp.float32)
    cnt_acc[...] += jnp.dot(onehot, jnp.ones((BN, 128), jnp.float32),
                            preferred_element_type=jnp.float32)

    @pl.when(i == NB - 1)
    def _():
        inv = 1.0 / jnp.maximum(cnt_acc[...], 1.0)
        scale = jnp.concatenate([inv] * 4, axis=1)
        pooled = pooled_acc[...] * scale
        o_ref[...] = (
            jnp.dot(pooled, wc_ref[...], preferred_element_type=jnp.float32)
            + bc_ref[...])


def _final_layer(h3, msg3, W1, b1, W2, b2, Wc_pad, bc_pad, batch3):
    return pl.pallas_call(
        _final_body,
        grid=(NB,),
        in_specs=[
            pl.BlockSpec((4, BN, 128), lambda i: (0, i, 0)),
            pl.BlockSpec((4, BN, 128), lambda i: (0, i, 0)),
            pl.BlockSpec((H, H), lambda i: (0, 0)),
            pl.BlockSpec((1, H), lambda i: (0, 0)),
            pl.BlockSpec((H, H), lambda i: (0, 0)),
            pl.BlockSpec((1, H), lambda i: (0, 0)),
            pl.BlockSpec((H, 128), lambda i: (0, 0)),
            pl.BlockSpec((1, 128), lambda i: (0, 0)),
            pl.BlockSpec((1, 1, BN), lambda i: (i, 0, 0)),
        ],
        out_specs=pl.BlockSpec((G, 128), lambda i: (0, 0)),
        out_shape=jax.ShapeDtypeStruct((G, 128), jnp.float32),
        scratch_shapes=[
            pltpu.VMEM((G, H), jnp.float32),
            pltpu.VMEM((G, 128), jnp.float32),
        ],
    )(h3, msg3, W1, b1.reshape(1, H), W2, b2.reshape(1, H),
      Wc_pad, bc_pad, batch3)


def kernel(x, edge_index, batch,
           W1_0, b1_0, W2_0, b2_0,
           W1_1, b1_1, W2_1, b2_1,
           W1_2, b1_2, W2_2, b2_2,
           W1_3, b1_3, W2_3, b2_3,
           W1_4, b1_4, W2_4, b2_4,
           Wc, bc):
    layers = [(W1_0, b1_0, W2_0, b2_0), (W1_1, b1_1, W2_1, b2_1),
              (W1_2, b1_2, W2_2, b2_2), (W1_3, b1_3, W2_3, b2_3),
              (W1_4, b1_4, W2_4, b2_4)]

    x_pad = jnp.pad(x, ((0, NP - N), (0, 0)))
    h3 = jnp.transpose(x_pad.reshape(NP, 2, 128), (1, 0, 2))
    src = jnp.concatenate(
        [edge_index[0], jnp.zeros((EP - E,), jnp.int32)]).reshape(EP // BLK, BLK)
    dst = jnp.concatenate(
        [edge_index[1], jnp.full((EP - E,), NP - 1, jnp.int32)]).reshape(EP // BLK, BLK)
    zeros_hbm = jnp.zeros((ZR, 128), jnp.float32)
    batch3 = jnp.concatenate(
        [batch, jnp.full((NP - N,), G, jnp.int32)]).reshape(NB, 1, BN)
    Wc_pad = jnp.pad(Wc, ((0, 0), (0, 128 - C)))
    bc_pad = jnp.pad(bc, ((0, 128 - C),)).reshape(1, 128)

    for l in range(L):
        W1, b1, W2, b2 = layers[l]
        nc = h3.shape[0]
        msg = _make_sc_segsum(nc)(h3.reshape(nc * NP, 128), src, dst, zeros_hbm)
        msg3 = msg.reshape(nc, NP, 128)
        if l < L - 1:
            h3 = _mlp_layer(h3, msg3, W1, b1, W2, b2)
        else:
            logits = _final_layer(h3, msg3, W1, b1, W2, b2,
                                  Wc_pad, bc_pad, batch3)
    return logits[:, :C]
